# baseline (speedup 1.0000x reference)
import functools

import jax
import jax.numpy as jnp
from jax import lax
from jax.experimental import pallas as pl
from jax.experimental.pallas import tpu as pltpu
from jax.experimental.pallas import tpu_sc as plsc

N_CLUSTERS = 512
DIM = 64
NC = 2
NS = 16
NW = NC * NS
UNROLL = 50


@functools.lru_cache(maxsize=None)
def _embed_lookup(NROW: int, NCOL: int):
    assert NROW % (NW * 2) == 0 and NCOL % UNROLL == 0
    rows_per_w = NROW // NW
    n_chunks = rows_per_w
    C = NCOL

    mesh = plsc.VectorSubcoreMesh(core_axis_name="c", subcore_axis_name="s")

    @functools.partial(
        pl.kernel,
        mesh=mesh,
        out_type=jax.ShapeDtypeStruct((NROW, NCOL, DIM), jnp.float32),
        scratch_types=[
            pltpu.VMEM_SHARED((N_CLUSTERS, 2 * DIM), jnp.float32),
            pltpu.VMEM((C,), jnp.int32),
            pltpu.VMEM((C,), jnp.int32),
            pltpu.VMEM((C, 2 * DIM), jnp.float32),
            pltpu.VMEM((C, 2 * DIM), jnp.float32),
            pltpu.VMEM((C, DIM), jnp.float32),
            pltpu.VMEM((C, DIM), jnp.float32),
            pltpu.SemaphoreType.DMA,
            pltpu.SemaphoreType.DMA,
            pltpu.SemaphoreType.DMA,
            pltpu.SemaphoreType.DMA,
            pltpu.SemaphoreType.DMA,
            pltpu.SemaphoreType.DMA,
        ],
    )
    def k(flat_hbm, table2_hbm, out_hbm, table_sh,
          idx_v0, idx_v1, wide_v0, wide_v1, rows_v0, rows_v1,
          isem0, isem1, osem0, osem1, gsem0, gsem1):
        idx_vs = [idx_v0, idx_v1]
        wide_vs = [wide_v0, wide_v1]
        rows_vs = [rows_v0, rows_v1]
        isems = [isem0, isem1]
        osems = [osem0, osem1]
        gsems = [gsem0, gsem1]
        sid = lax.axis_index("s")
        wid = sid * NC + lax.axis_index("c")
        rbase = wid * rows_per_w
        fbase = rbase * NCOL

        @pl.when(sid == 0)
        def _():
            for p in range(0, N_CLUSTERS, C):
                n = min(C, N_CLUSTERS - p)
                bounce = wide_v0.at[pl.ds(0, n)]
                pltpu.sync_copy(table2_hbm.at[pl.ds(p, n)], bounce)
                pltpu.sync_copy(bounce, table_sh.at[pl.ds(p, n)])

        plsc.subcore_barrier()

        for b in range(2):
            pltpu.async_copy(
                flat_hbm.at[pl.ds(fbase + b * C, C)], idx_vs[b], isems[b])
        pltpu.make_async_copy(
            flat_hbm.at[pl.ds(fbase, C)], idx_vs[0], isems[0]).wait()
        pltpu.async_copy(table_sh.at[idx_vs[0]], wide_vs[0], gsems[0])

        def step(i, b):
            b1 = 1 - b

            @pl.when(i >= 2)
            def _():
                pltpu.make_async_copy(
                    rows_vs[b], out_hbm.at[rbase], osems[b]).wait()

            pltpu.make_async_copy(
                table_sh.at[idx_vs[b]], wide_vs[b], gsems[b]).wait()

            @pl.when(i + 2 < n_chunks)
            def _():
                pltpu.async_copy(
                    flat_hbm.at[pl.ds(fbase + (i + 2) * C, C)],
                    idx_vs[b], isems[b])

            @pl.when(i + 1 < n_chunks)
            def _():
                pltpu.make_async_copy(
                    flat_hbm.at[pl.ds(fbase, C)], idx_vs[b1],
                    isems[b1]).wait()
                pltpu.async_copy(
                    table_sh.at[idx_vs[b1]], wide_vs[b1], gsems[b1])

            def repack(q, carry):
                for u in range(UNROLL):
                    r = q * UNROLL + u
                    for c in range(DIM // 16):
                        rows_vs[b][r, pl.ds(c * 16, 16)] = (
                            wide_vs[b][r, pl.ds(c * 16, 16)])
                return carry

            lax.fori_loop(0, C // UNROLL, repack, 0)

            pltpu.async_copy(rows_vs[b], out_hbm.at[rbase + i], osems[b])

        def outer(j, carry):
            step(2 * j, 0)
            step(2 * j + 1, 1)
            return carry

        lax.fori_loop(0, n_chunks // 2, outer, 0)

        for b in range(2):
            pltpu.make_async_copy(
                rows_vs[b], out_hbm.at[rbase], osems[b]).wait()

    return k


def kernel(x, table):
    flat = x.reshape(x.shape[0] * x.shape[1])
    table2 = jnp.concatenate([table, table], axis=1)
    return _embed_lookup(x.shape[0], x.shape[1])(flat, table2)
